# baseline (speedup 1.0000x reference)
import functools

import jax
import jax.numpy as jnp
from jax import lax
from jax.experimental import pallas as pl
from jax.experimental.pallas import tpu as pltpu
from jax.experimental.pallas import tpu_sc as plsc

NUM_EMB = 100000
DIM = 4
PDIM = 2 * DIM
ROWS_W = 16
NW = 32
L = 16
HALF = 1600


def _emb_kernel(b: int, s: int):
    rows_per_w = b // NW
    n_wnd = rows_per_w // ROWS_W
    assert n_wnd % 2 == 0
    W = ROWS_W * s
    assert W % HALF == 0
    stage_rows = (NUM_EMB // 16) // 8 * 8
    tail_rows = NUM_EMB - 16 * stage_rows

    mesh = plsc.VectorSubcoreMesh(core_axis_name="c", subcore_axis_name="s")

    @functools.partial(
        pl.kernel,
        mesh=mesh,
        out_type=jax.ShapeDtypeStruct((b, s * DIM), jnp.float32),
        scratch_types=[
            pltpu.VMEM_SHARED((NUM_EMB, PDIM), jnp.float32),
            pltpu.VMEM((ROWS_W, s), jnp.int32),
            pltpu.VMEM((ROWS_W, s), jnp.int32),
            pltpu.VMEM((W,), jnp.int32),
            pltpu.VMEM((W,), jnp.int32),
            pltpu.VMEM((W, PDIM), jnp.float32),
            pltpu.VMEM((W, PDIM), jnp.float32),
            pltpu.VMEM((ROWS_W, s * DIM), jnp.float32),
            pltpu.SemaphoreType.DMA,
            pltpu.SemaphoreType.DMA,
            pltpu.SemaphoreType.DMA,
            pltpu.SemaphoreType.DMA,
        ],
        compiler_params=pltpu.CompilerParams(
            use_tc_tiling_on_sc=False, needs_layout_passes=False),
    )
    def k(ids_hbm, table_hbm, out_hbm, table_sh,
          idx0, idx1, pidx0, pidx1, wide0, wide1, nar,
          isem0, isem1, gsem0, gsem1):
        cid = lax.axis_index("c")
        sid = lax.axis_index("s")
        nc = lax.axis_size("c")
        wid = sid * nc + cid
        row0 = wid * rows_per_w

        r0 = sid * stage_rows
        pltpu.sync_copy(
            table_hbm.at[pl.ds(r0, stage_rows)],
            table_sh.at[pl.ds(r0, stage_rows)],
        )

        @pl.when(sid == 15)
        def _():
            t0 = 16 * stage_rows
            pltpu.sync_copy(
                table_hbm.at[pl.ds(t0, tail_rows)],
                table_sh.at[pl.ds(t0, tail_rows)],
            )

        plsc.subcore_barrier()

        idxb = (idx0, idx1)
        pidxb = (pidx0, pidx1)
        wideb = (wide0, wide1)
        isems = (isem0, isem1)
        gsems = (gsem0, gsem1)

        lane = lax.iota(jnp.int32, L)
        rowp = lax.shift_right_logical(lane, 2)
        colp = lax.bitwise_and(lane, 3)

        def fire_gathers(buf):
            for h in range(W // HALF):
                pltpu.async_copy(
                    table_hbm.at[pidxb[buf].at[pl.ds(h * HALF, HALF)]],
                    wideb[buf].at[pl.ds(h * HALF, HALF)],
                    gsems[buf])

        def drain_gathers(buf):
            for h in range(W // HALF):
                pltpu.make_async_copy(
                    table_hbm.at[pidxb[buf].at[pl.ds(h * HALF, HALF)]],
                    wideb[buf].at[pl.ds(h * HALF, HALF)],
                    gsems[buf]).wait()

        def split_pass(buf):
            def sp(q, carry):
                for u in range(4):
                    t = 4 * q + u
                    p = L * t + lane
                    rows = lax.div(p, s)
                    cols = p - s * rows
                    v = plsc.load_gather(idxb[buf], [rows, cols])
                    pidxb[buf][pl.ds(L * t, L)] = v
                return carry
            lax.fori_loop(0, W // (L * 4), sp, 0)

        vregs_row = s * DIM // L

        def compact_pass(buf, wnd):
            def cp_r(r, carry):
                def cp(q, carry2):
                    for u in range(2):
                        j = 2 * q + u
                        v = r * vregs_row + j
                        rows = rowp + 4 * v
                        g = plsc.load_gather(wideb[buf], [rows, colp])
                        nar[r, pl.ds(L * j, L)] = g
                    return carry2
                lax.fori_loop(0, vregs_row // 2, cp, 0)
                return carry
            lax.fori_loop(0, ROWS_W, cp_r, 0)
            pltpu.sync_copy(
                nar,
                out_hbm.at[pl.ds(row0 + wnd * ROWS_W, ROWS_W)])

        pltpu.sync_copy(ids_hbm.at[pl.ds(row0, ROWS_W)], idx0)
        split_pass(0)
        fire_gathers(0)
        pltpu.async_copy(
            ids_hbm.at[pl.ds(row0 + ROWS_W, ROWS_W)], idx1, isem1)

        def body(it, carry):
            for kk in (0, 1):
                wnd = 2 * it + kk
                buf = kk
                nbuf = 1 - kk

                @pl.when(wnd + 1 < n_wnd)
                def _():
                    pltpu.make_async_copy(
                        ids_hbm.at[pl.ds(row0 + (wnd + 1) * ROWS_W, ROWS_W)],
                        idxb[nbuf], isems[nbuf]).wait()
                    split_pass(nbuf)
                    fire_gathers(nbuf)

                @pl.when(wnd + 2 < n_wnd)
                def _():
                    pltpu.async_copy(
                        ids_hbm.at[pl.ds(row0 + (wnd + 2) * ROWS_W, ROWS_W)],
                        idxb[buf], isems[buf])

                drain_gathers(buf)
                compact_pass(buf, wnd)
            return carry

        lax.fori_loop(0, n_wnd // 2, body, 0)

    return k


def kernel(src_input_ids, src_attention_mask, emb_table):
    del src_attention_mask
    b, s = src_input_ids.shape
    assert b % (NW * ROWS_W) == 0
    ids = src_input_ids.astype(jnp.int32)
    table_dup = jnp.concatenate([emb_table, emb_table], axis=1)
    out = _emb_kernel(b, s)(ids, table_dup)
    return out.reshape(b, s, DIM)


# speedup vs baseline: 1.0138x; 1.0138x over previous
import functools

import jax
import jax.numpy as jnp
from jax import lax
from jax.experimental import pallas as pl
from jax.experimental.pallas import tpu as pltpu
from jax.experimental.pallas import tpu_sc as plsc

NUM_EMB = 100000
DIM = 4
PDIM = 2 * DIM
ROWS_W = 16
NW = 32
L = 16
HALF = 1600


def _emb_kernel(b: int, s: int):
    rows_per_w = b // NW
    n_wnd = rows_per_w // ROWS_W
    assert n_wnd % 2 == 0
    W = ROWS_W * s
    assert W % HALF == 0
    stage_rows = (NUM_EMB // 16) // 8 * 8
    tail_rows = NUM_EMB - 16 * stage_rows

    mesh = plsc.VectorSubcoreMesh(core_axis_name="c", subcore_axis_name="s")

    @functools.partial(
        pl.kernel,
        mesh=mesh,
        out_type=jax.ShapeDtypeStruct((b, s * DIM), jnp.float32),
        scratch_types=[
            pltpu.VMEM_SHARED((NUM_EMB, PDIM), jnp.float32),
            pltpu.VMEM((ROWS_W, s), jnp.int32),
            pltpu.VMEM((ROWS_W, s), jnp.int32),
            pltpu.VMEM((W,), jnp.int32),
            pltpu.VMEM((W,), jnp.int32),
            pltpu.VMEM((W, PDIM), jnp.float32),
            pltpu.VMEM((W, PDIM), jnp.float32),
            pltpu.VMEM((ROWS_W, s * DIM), jnp.float32),
            pltpu.SemaphoreType.DMA,
            pltpu.SemaphoreType.DMA,
            pltpu.SemaphoreType.DMA,
            pltpu.SemaphoreType.DMA,
        ],
        compiler_params=pltpu.CompilerParams(
            use_tc_tiling_on_sc=False, needs_layout_passes=False),
    )
    def k(ids_hbm, table_hbm, out_hbm, table_sh,
          idx0, idx1, pidx0, pidx1, wide0, wide1, nar,
          isem0, isem1, gsem0, gsem1):
        cid = lax.axis_index("c")
        sid = lax.axis_index("s")
        nc = lax.axis_size("c")
        wid = sid * nc + cid
        row0 = wid * rows_per_w

        r0 = sid * stage_rows
        pltpu.sync_copy(
            table_hbm.at[pl.ds(r0, stage_rows)],
            table_sh.at[pl.ds(r0, stage_rows)],
        )

        @pl.when(sid == 15)
        def _():
            t0 = 16 * stage_rows
            pltpu.sync_copy(
                table_hbm.at[pl.ds(t0, tail_rows)],
                table_sh.at[pl.ds(t0, tail_rows)],
            )

        plsc.subcore_barrier()

        idxb = (idx0, idx1)
        pidxb = (pidx0, pidx1)
        wideb = (wide0, wide1)
        isems = (isem0, isem1)
        gsems = (gsem0, gsem1)

        lane = lax.iota(jnp.int32, L)
        rowp = lax.shift_right_logical(lane, 2)
        colp = lax.bitwise_and(lane, 3)

        def fire_gathers(buf):
            for h in range(W // HALF):
                pltpu.async_copy(
                    table_sh.at[pidxb[buf].at[pl.ds(h * HALF, HALF)]],
                    wideb[buf].at[pl.ds(h * HALF, HALF)],
                    gsems[buf])

        def drain_gathers(buf):
            for h in range(W // HALF):
                pltpu.make_async_copy(
                    table_sh.at[pidxb[buf].at[pl.ds(h * HALF, HALF)]],
                    wideb[buf].at[pl.ds(h * HALF, HALF)],
                    gsems[buf]).wait()

        def split_pass(buf):
            def sp(q, carry):
                for u in range(4):
                    t = 4 * q + u
                    p = L * t + lane
                    rows = lax.div(p, s)
                    cols = p - s * rows
                    v = plsc.load_gather(idxb[buf], [rows, cols])
                    pidxb[buf][pl.ds(L * t, L)] = v
                return carry
            lax.fori_loop(0, W // (L * 4), sp, 0)

        vregs_row = s * DIM // L

        def compact_pass(buf, wnd):
            def cp_r(r, carry):
                def cp(q, carry2):
                    for u in range(2):
                        j = 2 * q + u
                        v = r * vregs_row + j
                        rows = rowp + 4 * v
                        g = plsc.load_gather(wideb[buf], [rows, colp])
                        nar[r, pl.ds(L * j, L)] = g
                    return carry2
                lax.fori_loop(0, vregs_row // 2, cp, 0)
                return carry
            lax.fori_loop(0, ROWS_W, cp_r, 0)
            pltpu.sync_copy(
                nar,
                out_hbm.at[pl.ds(row0 + wnd * ROWS_W, ROWS_W)])

        pltpu.sync_copy(ids_hbm.at[pl.ds(row0, ROWS_W)], idx0)
        split_pass(0)
        fire_gathers(0)
        pltpu.async_copy(
            ids_hbm.at[pl.ds(row0 + ROWS_W, ROWS_W)], idx1, isem1)

        def body(it, carry):
            for kk in (0, 1):
                wnd = 2 * it + kk
                buf = kk
                nbuf = 1 - kk

                @pl.when(wnd + 1 < n_wnd)
                def _():
                    pltpu.make_async_copy(
                        ids_hbm.at[pl.ds(row0 + (wnd + 1) * ROWS_W, ROWS_W)],
                        idxb[nbuf], isems[nbuf]).wait()
                    split_pass(nbuf)
                    fire_gathers(nbuf)

                @pl.when(wnd + 2 < n_wnd)
                def _():
                    pltpu.async_copy(
                        ids_hbm.at[pl.ds(row0 + (wnd + 2) * ROWS_W, ROWS_W)],
                        idxb[buf], isems[buf])

                drain_gathers(buf)
                compact_pass(buf, wnd)
            return carry

        lax.fori_loop(0, n_wnd // 2, body, 0)

    return k


def kernel(src_input_ids, src_attention_mask, emb_table):
    del src_attention_mask
    b, s = src_input_ids.shape
    assert b % (NW * ROWS_W) == 0
    ids = src_input_ids.astype(jnp.int32)
    table_dup = jnp.concatenate([emb_table, emb_table], axis=1)
    out = _emb_kernel(b, s)(ids, table_dup)
    return out.reshape(b, s, DIM)
